# baseline (speedup 1.0000x reference)
import functools

import jax
import jax.numpy as jnp
from jax import lax
from jax.experimental import pallas as pl
from jax.experimental.pallas import tpu as pltpu
from jax.experimental.pallas import tpu_sc as plsc

N = 10000
E = 320000
D_IN = 128
D = 256
NP = 10240
RPT = NP // 16
CHUNK = 128
EPAD = 327680
NBUF = 2
BLKE = 2048
BCH = BLKE // CHUNK

_mesh = plsc.VectorSubcoreMesh(core_axis_name="c", subcore_axis_name="s")


def _edge_pipeline(table_hbm, src_hbm, dst2d_hbm, src_base, blk0, nblk,
                   sbuf, dbuf, rows, acc, si, sg, ss, extra_scatter=None,
                   extra_wait=None):
    def i_start(j, jb):
        pltpu.async_copy(src_hbm.at[pl.ds(src_base + j * BLKE, BLKE)],
                         sbuf[jb], si[jb])
        pltpu.async_copy(dst2d_hbm.at[pl.ds(blk0 + j * BCH, BCH)],
                         dbuf[jb], si[jb])

    def i_wait(jb):
        pltpu.make_async_copy(src_hbm.at[pl.ds(0, BLKE)], sbuf[jb],
                              si[jb]).wait()
        pltpu.make_async_copy(dst2d_hbm.at[pl.ds(0, BCH)], dbuf[jb],
                              si[jb]).wait()

    def g_start(jb, t, b):
        pltpu.async_copy(
            table_hbm.at[sbuf[jb].at[pl.ds(t * CHUNK, CHUNK)]], rows[b],
            sg[b])

    def g_wait(b):
        pltpu.make_async_copy(table_hbm.at[sbuf[0].at[pl.ds(0, CHUNK)]],
                              rows[b], sg[b]).wait()

    def s_start(jb, t, b):
        pltpu.async_copy(rows[b], acc.at[dbuf[jb].at[t]], ss[b], add=True)
        if extra_scatter is not None:
            extra_scatter(jb, t, b)

    def s_wait(b):
        pltpu.make_async_copy(rows[b], acc.at[dbuf[0].at[0]], ss[b]).wait()
        if extra_wait is not None:
            extra_wait(b)

    i_start(0, 0)
    i_wait(0)
    if nblk > 1:
        i_start(1, 1)

    for j in range(nblk):
        jb = j % 2

        for b in range(NBUF):
            g_start(jb, b, b)

        def chunk(t2, _, jb=jb):
            for b in range(NBUF):
                t = t2 * NBUF + b
                g_wait(b)
                s_start(jb, t, b)
                s_wait(b)

                @pl.when(t + NBUF < BCH)
                def _(t=t, b=b, jb=jb):
                    g_start(jb, t + NBUF, b)
            return 0

        lax.fori_loop(0, BCH // NBUF, chunk, 0)

        if j + 2 < nblk:
            i_start(j + 2, jb)
        if j + 1 < nblk:
            i_wait((j + 1) % 2)


def _agg0_body(x_hbm, src_hbm, dst2d_hbm, z2d_hbm, z1d_hbm,
               agg_hbm, cnt_hbm,
               sbuf0, sbuf1, dbuf0, dbuf1, ones_v, rows0, rows1,
               acc, cacc,
               si0, si1, sg0, sg1, ss0, ss1, sc0, sc1):
    c = lax.axis_index("c")
    s = lax.axis_index("s")
    w = c * 16 + s
    nblk = EPAD // 32 // BLKE

    r0 = s * RPT
    pltpu.sync_copy(z2d_hbm.at[pl.ds(r0, RPT)], acc.at[pl.ds(r0, RPT)])
    pltpu.sync_copy(z1d_hbm.at[pl.ds(r0, RPT)], cacc.at[pl.ds(r0, RPT)])
    o = jnp.ones((16,), jnp.float32)
    for k in range(CHUNK // 16):
        ones_v[pl.ds(k * 16, 16)] = o
    plsc.subcore_barrier()

    sc = [sc0, sc1]
    dbuf = [dbuf0, dbuf1]

    def cnt_scatter(jb, t, b):
        pltpu.async_copy(ones_v, cacc.at[dbuf[jb].at[t]], sc[b], add=True)

    def cnt_wait(b):
        pltpu.make_async_copy(ones_v, cacc.at[dbuf0.at[0]], sc[b]).wait()

    _edge_pipeline(x_hbm, src_hbm, dst2d_hbm,
                   src_base=w * (EPAD // 32), blk0=w * (EPAD // 32 // CHUNK),
                   nblk=nblk,
                   sbuf=[sbuf0, sbuf1], dbuf=dbuf,
                   rows=[rows0, rows1], acc=acc,
                   si=[si0, si1], sg=[sg0, sg1], ss=[ss0, ss1],
                   extra_scatter=cnt_scatter, extra_wait=cnt_wait)

    plsc.subcore_barrier()
    pltpu.sync_copy(acc.at[pl.ds(r0, RPT)],
                    agg_hbm.at[pl.ds(c * NP + r0, RPT)])
    pltpu.sync_copy(cacc.at[pl.ds(r0, RPT)],
                    cnt_hbm.at[pl.ds(c * NP + r0, RPT)])


_sc_agg0 = pl.kernel(
    _agg0_body,
    out_type=(jax.ShapeDtypeStruct((2 * NP, 128), jnp.float32),
              jax.ShapeDtypeStruct((2 * NP,), jnp.float32)),
    mesh=_mesh,
    scratch_types=[
        pltpu.VMEM((BLKE,), jnp.int32),
        pltpu.VMEM((BLKE,), jnp.int32),
        pltpu.VMEM((BCH, CHUNK), jnp.int32),
        pltpu.VMEM((BCH, CHUNK), jnp.int32),
        pltpu.VMEM((CHUNK,), jnp.float32),
        pltpu.VMEM((CHUNK, 128), jnp.float32),
        pltpu.VMEM((CHUNK, 128), jnp.float32),
        pltpu.VMEM_SHARED((NP, 128), jnp.float32),
        pltpu.VMEM_SHARED((NP,), jnp.float32),
    ] + [pltpu.SemaphoreType.DMA] * 8,
)


def _agg_body(h_hbm, srcb_hbm, dst2d_hbm, z2d_hbm,
              agg_hbm,
              sbuf0, sbuf1, dbuf0, dbuf1, rows0, rows1,
              acc,
              si0, si1, sg0, sg1, ss0, ss1):
    c = lax.axis_index("c")
    s = lax.axis_index("s")
    ept = EPAD // 16
    nblk = ept // BLKE

    r0 = s * RPT
    pltpu.sync_copy(z2d_hbm.at[pl.ds(r0, RPT)], acc.at[pl.ds(r0, RPT)])
    plsc.subcore_barrier()

    _edge_pipeline(h_hbm, srcb_hbm, dst2d_hbm,
                   src_base=c * EPAD + s * ept, blk0=s * (ept // CHUNK),
                   nblk=nblk,
                   sbuf=[sbuf0, sbuf1], dbuf=[dbuf0, dbuf1],
                   rows=[rows0, rows1], acc=acc,
                   si=[si0, si1], sg=[sg0, sg1], ss=[ss0, ss1])

    plsc.subcore_barrier()
    pltpu.sync_copy(acc.at[pl.ds(r0, RPT)],
                    agg_hbm.at[pl.ds(c * NP + r0, RPT)])


_sc_agg = pl.kernel(
    _agg_body,
    out_type=jax.ShapeDtypeStruct((2 * NP, 128), jnp.float32),
    mesh=_mesh,
    scratch_types=[
        pltpu.VMEM((BLKE,), jnp.int32),
        pltpu.VMEM((BLKE,), jnp.int32),
        pltpu.VMEM((BCH, CHUNK), jnp.int32),
        pltpu.VMEM((BCH, CHUNK), jnp.int32),
        pltpu.VMEM((CHUNK, 128), jnp.float32),
        pltpu.VMEM((CHUNK, 128), jnp.float32),
        pltpu.VMEM_SHARED((NP, 128), jnp.float32),
    ] + [pltpu.SemaphoreType.DMA] * 6,
)


ROW_BLK = 2048


def _tc_layer_body(nparts_in, parts_out,
                   agg_ref, cnt_ref, h_ref, wl_ref, wr_ref, b_ref, o_ref):
    cnt = cnt_ref[0] + cnt_ref[1]
    inv = 1.0 / jnp.maximum(cnt, 1.0)
    acc = jnp.zeros((ROW_BLK, 128), jnp.float32)
    for c in range(2):
        acc = acc + jnp.dot(agg_ref[c] * inv[:, None], wl_ref[c],
                            preferred_element_type=jnp.float32)
    for q in range(nparts_in):
        acc = acc + jnp.dot(h_ref[q], wr_ref[q],
                            preferred_element_type=jnp.float32)
    acc = acc + b_ref[0][None, :]
    out = jnp.maximum(acc, 0.0)
    if parts_out:
        o_ref[...] = out[None]
    else:
        o_ref[...] = out


def _tc_layer(agg, cnt, h_parts, wlt, wrt, b, parts_out):
    p_in = h_parts.shape[0]
    grid = (5, 2)
    if parts_out:
        out_shape = jax.ShapeDtypeStruct((2, NP, 128), jnp.float32)
        out_spec = pl.BlockSpec((1, ROW_BLK, 128), lambda i, p: (p, i, 0))
    else:
        out_shape = jax.ShapeDtypeStruct((N, D), jnp.float32)
        out_spec = pl.BlockSpec((ROW_BLK, 128), lambda i, p: (i, p))
    return pl.pallas_call(
        functools.partial(_tc_layer_body, p_in, parts_out),
        grid=grid,
        in_specs=[
            pl.BlockSpec((2, ROW_BLK, 128), lambda i, p: (0, i, 0)),
            pl.BlockSpec((2, ROW_BLK), lambda i, p: (0, i)),
            pl.BlockSpec((p_in, ROW_BLK, 128), lambda i, p: (0, i, 0)),
            pl.BlockSpec((2, 128, 128), lambda i, p: (0, 0, p)),
            pl.BlockSpec((p_in, 128, 128), lambda i, p: (0, 0, p)),
            pl.BlockSpec((1, 128), lambda i, p: (0, p)),
        ],
        out_specs=out_spec,
        out_shape=out_shape,
    )(agg, cnt, h_parts, wlt, wrt, b)


def kernel(x, edge_index, Wl0, Wr0, b0, Wl1, Wr1, b1, Wl2, Wr2, b2):
    src = edge_index[0]
    dst = edge_index[1]
    npad = EPAD - E
    src_pad = jnp.concatenate([src, jnp.zeros((npad,), jnp.int32)])
    dst_pad = jnp.concatenate([dst, jnp.full((npad,), NP - 1, jnp.int32)])
    dst2d = dst_pad.reshape(EPAD // CHUNK, CHUNK)
    srcb = jnp.concatenate([src_pad, src_pad + NP])
    z2d = jnp.zeros((NP, 128), jnp.float32)
    z1d = jnp.zeros((NP,), jnp.float32)

    agg0, cnt = _sc_agg0(x, src_pad, dst2d, z2d, z1d)
    agg0 = agg0.reshape(2, NP, 128)
    cnt = cnt.reshape(2, NP)
    h1 = _tc_layer(agg0, cnt, x.reshape(1, N, 128),
                   jnp.stack([Wl0.T, Wl0.T]), Wr0.T.reshape(1, 128, D),
                   b0.reshape(1, D), parts_out=True)

    agg1 = _sc_agg(h1.reshape(2 * NP, 128), srcb, dst2d, z2d)
    h2 = _tc_layer(agg1.reshape(2, NP, 128), cnt, h1,
                   Wl1.T.reshape(2, 128, D), Wr1.T.reshape(2, 128, D),
                   b1.reshape(1, D), parts_out=True)

    agg2 = _sc_agg(h2.reshape(2 * NP, 128), srcb, dst2d, z2d)
    h3 = _tc_layer(agg2.reshape(2, NP, 128), cnt, h2,
                   Wl2.T.reshape(2, 128, D), Wr2.T.reshape(2, 128, D),
                   b2.reshape(1, D), parts_out=False)

    return h3.reshape(1, N, D)


# speedup vs baseline: 1.0200x; 1.0200x over previous
import functools

import jax
import jax.numpy as jnp
from jax import lax
from jax.experimental import pallas as pl
from jax.experimental.pallas import tpu as pltpu
from jax.experimental.pallas import tpu_sc as plsc

N = 10000
E = 320000
D_IN = 128
D = 256
NP = 10240
RPT = NP // 16
CHUNK = 128
EPAD = 327680
NBUF = 2
BLKE = 2048
BCH = BLKE // CHUNK

_mesh = plsc.VectorSubcoreMesh(core_axis_name="c", subcore_axis_name="s")


def _edge_pipeline(table_hbm, src_hbm, dst2d_hbm, src_base, blk0, nblk,
                   sbuf, dbuf, rows, acc, si, sg, ss, extra_scatter=None,
                   extra_wait=None):
    def i_start(j, jb):
        pltpu.async_copy(src_hbm.at[pl.ds(src_base + j * BLKE, BLKE)],
                         sbuf[jb], si[jb])
        pltpu.async_copy(dst2d_hbm.at[pl.ds(blk0 + j * BCH, BCH)],
                         dbuf[jb], si[jb])

    def i_wait(jb):
        pltpu.make_async_copy(src_hbm.at[pl.ds(0, BLKE)], sbuf[jb],
                              si[jb]).wait()
        pltpu.make_async_copy(dst2d_hbm.at[pl.ds(0, BCH)], dbuf[jb],
                              si[jb]).wait()

    def g_start(jb, t, b):
        pltpu.async_copy(
            table_hbm.at[sbuf[jb].at[pl.ds(t * CHUNK, CHUNK)]], rows[b],
            sg[b])

    def g_wait(b):
        pltpu.make_async_copy(table_hbm.at[sbuf[0].at[pl.ds(0, CHUNK)]],
                              rows[b], sg[b]).wait()

    def s_start(jb, t, b):
        pltpu.async_copy(rows[b], acc.at[dbuf[jb].at[t]], ss[b], add=True)
        if extra_scatter is not None:
            extra_scatter(jb, t, b)

    def s_wait(b):
        pltpu.make_async_copy(rows[b], acc.at[dbuf[0].at[0]], ss[b]).wait()
        if extra_wait is not None:
            extra_wait(b)

    i_start(0, 0)
    i_wait(0)
    if nblk > 1:
        i_start(1, 1)

    for b in range(NBUF):
        g_start(0, b, b)

    for j in range(nblk):
        jb = j % 2

        def chunk(t2, _, jb=jb):
            for b in range(NBUF):
                t = t2 * NBUF + b
                g_wait(b)
                s_start(jb, t, b)
                s_wait(b)
                g_start(jb, t + NBUF, b)
            return 0

        lax.fori_loop(0, BCH // NBUF - 1, chunk, 0)

        if j + 1 < nblk:
            i_wait((j + 1) % 2)

        for b in range(NBUF):
            t = BCH - NBUF + b
            g_wait(b)
            s_start(jb, t, b)
            s_wait(b)
            if j + 1 < nblk:
                g_start(1 - jb, b, b)

        if j + 2 < nblk:
            i_start(j + 2, jb)


def _agg0_body(x_hbm, src_hbm, dst2d_hbm, z2d_hbm, z1d_hbm,
               agg_hbm, cnt_hbm,
               sbuf0, sbuf1, dbuf0, dbuf1, ones_v, rows0, rows1,
               acc, cacc,
               si0, si1, sg0, sg1, ss0, ss1, sc0, sc1):
    c = lax.axis_index("c")
    s = lax.axis_index("s")
    w = c * 16 + s
    nblk = EPAD // 32 // BLKE

    r0 = s * RPT
    pltpu.sync_copy(z2d_hbm.at[pl.ds(r0, RPT)], acc.at[pl.ds(r0, RPT)])
    pltpu.sync_copy(z1d_hbm.at[pl.ds(r0, RPT)], cacc.at[pl.ds(r0, RPT)])
    o = jnp.ones((16,), jnp.float32)
    for k in range(CHUNK // 16):
        ones_v[pl.ds(k * 16, 16)] = o
    plsc.subcore_barrier()

    sc = [sc0, sc1]
    dbuf = [dbuf0, dbuf1]

    def cnt_scatter(jb, t, b):
        pltpu.async_copy(ones_v, cacc.at[dbuf[jb].at[t]], sc[b], add=True)

    def cnt_wait(b):
        pltpu.make_async_copy(ones_v, cacc.at[dbuf0.at[0]], sc[b]).wait()

    _edge_pipeline(x_hbm, src_hbm, dst2d_hbm,
                   src_base=w * (EPAD // 32), blk0=w * (EPAD // 32 // CHUNK),
                   nblk=nblk,
                   sbuf=[sbuf0, sbuf1], dbuf=dbuf,
                   rows=[rows0, rows1], acc=acc,
                   si=[si0, si1], sg=[sg0, sg1], ss=[ss0, ss1],
                   extra_scatter=cnt_scatter, extra_wait=cnt_wait)

    plsc.subcore_barrier()
    pltpu.sync_copy(acc.at[pl.ds(r0, RPT)],
                    agg_hbm.at[pl.ds(c * NP + r0, RPT)])
    pltpu.sync_copy(cacc.at[pl.ds(r0, RPT)],
                    cnt_hbm.at[pl.ds(c * NP + r0, RPT)])


_sc_agg0 = pl.kernel(
    _agg0_body,
    out_type=(jax.ShapeDtypeStruct((2 * NP, 128), jnp.float32),
              jax.ShapeDtypeStruct((2 * NP,), jnp.float32)),
    mesh=_mesh,
    scratch_types=[
        pltpu.VMEM((BLKE,), jnp.int32),
        pltpu.VMEM((BLKE,), jnp.int32),
        pltpu.VMEM((BCH, CHUNK), jnp.int32),
        pltpu.VMEM((BCH, CHUNK), jnp.int32),
        pltpu.VMEM((CHUNK,), jnp.float32),
        pltpu.VMEM((CHUNK, 128), jnp.float32),
        pltpu.VMEM((CHUNK, 128), jnp.float32),
        pltpu.VMEM_SHARED((NP, 128), jnp.float32),
        pltpu.VMEM_SHARED((NP,), jnp.float32),
    ] + [pltpu.SemaphoreType.DMA] * 8,
)


def _agg_body(h_hbm, srcb_hbm, dst2d_hbm, z2d_hbm,
              agg_hbm,
              sbuf0, sbuf1, dbuf0, dbuf1, rows0, rows1,
              acc,
              si0, si1, sg0, sg1, ss0, ss1):
    c = lax.axis_index("c")
    s = lax.axis_index("s")
    ept = EPAD // 16
    nblk = ept // BLKE

    r0 = s * RPT
    pltpu.sync_copy(z2d_hbm.at[pl.ds(r0, RPT)], acc.at[pl.ds(r0, RPT)])
    plsc.subcore_barrier()

    _edge_pipeline(h_hbm, srcb_hbm, dst2d_hbm,
                   src_base=c * EPAD + s * ept, blk0=s * (ept // CHUNK),
                   nblk=nblk,
                   sbuf=[sbuf0, sbuf1], dbuf=[dbuf0, dbuf1],
                   rows=[rows0, rows1], acc=acc,
                   si=[si0, si1], sg=[sg0, sg1], ss=[ss0, ss1])

    plsc.subcore_barrier()
    pltpu.sync_copy(acc.at[pl.ds(r0, RPT)],
                    agg_hbm.at[pl.ds(c * NP + r0, RPT)])


_sc_agg = pl.kernel(
    _agg_body,
    out_type=jax.ShapeDtypeStruct((2 * NP, 128), jnp.float32),
    mesh=_mesh,
    scratch_types=[
        pltpu.VMEM((BLKE,), jnp.int32),
        pltpu.VMEM((BLKE,), jnp.int32),
        pltpu.VMEM((BCH, CHUNK), jnp.int32),
        pltpu.VMEM((BCH, CHUNK), jnp.int32),
        pltpu.VMEM((CHUNK, 128), jnp.float32),
        pltpu.VMEM((CHUNK, 128), jnp.float32),
        pltpu.VMEM_SHARED((NP, 128), jnp.float32),
    ] + [pltpu.SemaphoreType.DMA] * 6,
)


ROW_BLK = 2048


def _tc_layer_body(nparts_in, parts_out,
                   agg_ref, cnt_ref, h_ref, wl_ref, wr_ref, b_ref, o_ref):
    cnt = cnt_ref[0] + cnt_ref[1]
    inv = 1.0 / jnp.maximum(cnt, 1.0)
    acc = jnp.zeros((ROW_BLK, 128), jnp.float32)
    for c in range(2):
        acc = acc + jnp.dot(agg_ref[c] * inv[:, None], wl_ref[c],
                            preferred_element_type=jnp.float32)
    for q in range(nparts_in):
        acc = acc + jnp.dot(h_ref[q], wr_ref[q],
                            preferred_element_type=jnp.float32)
    acc = acc + b_ref[0][None, :]
    out = jnp.maximum(acc, 0.0)
    if parts_out:
        o_ref[...] = out[None]
    else:
        o_ref[...] = out


def _tc_layer(agg, cnt, h_parts, wlt, wrt, b, parts_out):
    p_in = h_parts.shape[0]
    grid = (5, 2)
    if parts_out:
        out_shape = jax.ShapeDtypeStruct((2, NP, 128), jnp.float32)
        out_spec = pl.BlockSpec((1, ROW_BLK, 128), lambda i, p: (p, i, 0))
    else:
        out_shape = jax.ShapeDtypeStruct((N, D), jnp.float32)
        out_spec = pl.BlockSpec((ROW_BLK, 128), lambda i, p: (i, p))
    return pl.pallas_call(
        functools.partial(_tc_layer_body, p_in, parts_out),
        grid=grid,
        in_specs=[
            pl.BlockSpec((2, ROW_BLK, 128), lambda i, p: (0, i, 0)),
            pl.BlockSpec((2, ROW_BLK), lambda i, p: (0, i)),
            pl.BlockSpec((p_in, ROW_BLK, 128), lambda i, p: (0, i, 0)),
            pl.BlockSpec((2, 128, 128), lambda i, p: (0, 0, p)),
            pl.BlockSpec((p_in, 128, 128), lambda i, p: (0, 0, p)),
            pl.BlockSpec((1, 128), lambda i, p: (0, p)),
        ],
        out_specs=out_spec,
        out_shape=out_shape,
    )(agg, cnt, h_parts, wlt, wrt, b)


def kernel(x, edge_index, Wl0, Wr0, b0, Wl1, Wr1, b1, Wl2, Wr2, b2):
    src = edge_index[0]
    dst = edge_index[1]
    npad = EPAD - E
    src_pad = jnp.concatenate([src, jnp.zeros((npad,), jnp.int32)])
    dst_pad = jnp.concatenate([dst, jnp.full((npad,), NP - 1, jnp.int32)])
    dst2d = dst_pad.reshape(EPAD // CHUNK, CHUNK)
    srcb = jnp.concatenate([src_pad, src_pad + NP])
    z2d = jnp.zeros((NP, 128), jnp.float32)
    z1d = jnp.zeros((NP,), jnp.float32)

    agg0, cnt = _sc_agg0(x, src_pad, dst2d, z2d, z1d)
    agg0 = agg0.reshape(2, NP, 128)
    cnt = cnt.reshape(2, NP)
    h1 = _tc_layer(agg0, cnt, x.reshape(1, N, 128),
                   jnp.stack([Wl0.T, Wl0.T]), Wr0.T.reshape(1, 128, D),
                   b0.reshape(1, D), parts_out=True)

    agg1 = _sc_agg(h1.reshape(2 * NP, 128), srcb, dst2d, z2d)
    h2 = _tc_layer(agg1.reshape(2, NP, 128), cnt, h1,
                   Wl1.T.reshape(2, 128, D), Wr1.T.reshape(2, 128, D),
                   b1.reshape(1, D), parts_out=True)

    agg2 = _sc_agg(h2.reshape(2 * NP, 128), srcb, dst2d, z2d)
    h3 = _tc_layer(agg2.reshape(2, NP, 128), cnt, h2,
                   Wl2.T.reshape(2, 128, D), Wr2.T.reshape(2, 128, D),
                   b2.reshape(1, D), parts_out=False)

    return h3.reshape(1, N, D)


# speedup vs baseline: 1.0223x; 1.0022x over previous
import functools

import jax
import jax.numpy as jnp
from jax import lax
from jax.experimental import pallas as pl
from jax.experimental.pallas import tpu as pltpu
from jax.experimental.pallas import tpu_sc as plsc

N = 10000
E = 320000
D_IN = 128
D = 256
NP = 10240
RPT = NP // 16
CHUNK = 128
EPAD = 327680
NBUF = 2
BLKE = 2048
BCH = BLKE // CHUNK

_mesh = plsc.VectorSubcoreMesh(core_axis_name="c", subcore_axis_name="s")


NIDX = 4


def _edge_pipeline(table_hbm, src_hbm, dst_hbm, src_base, dst_base, nchunk,
                   idxv, dstv, rows, acc, six, sid, sg, ss,
                   extra_scatter=None, extra_wait=None):
    def isx(t, q):
        pltpu.async_copy(src_hbm.at[pl.ds(src_base + t * CHUNK, CHUNK)],
                         idxv[q], six[q])

    def iwx(q):
        pltpu.make_async_copy(src_hbm.at[pl.ds(0, CHUNK)], idxv[q],
                              six[q]).wait()

    def isd(t, q):
        pltpu.async_copy(dst_hbm.at[pl.ds(dst_base + t * CHUNK, CHUNK)],
                         dstv[q], sid[q])

    def iwd(q):
        pltpu.make_async_copy(dst_hbm.at[pl.ds(0, CHUNK)], dstv[q],
                              sid[q]).wait()

    def gs(t_q, b):
        pltpu.async_copy(table_hbm.at[idxv[t_q]], rows[b], sg[b])

    def gw(b):
        pltpu.make_async_copy(table_hbm.at[idxv[0]], rows[b], sg[b]).wait()

    def ss_(q, b):
        pltpu.async_copy(rows[b], acc.at[dstv[q]], ss[b], add=True)
        if extra_scatter is not None:
            extra_scatter(q, b)

    def sw(b):
        pltpu.make_async_copy(rows[b], acc.at[dstv[0]], ss[b]).wait()
        if extra_wait is not None:
            extra_wait(b)

    for q in range(NIDX):
        isx(q, q)
        isd(q, q)
    for b in range(NBUF):
        iwx(b)
        gs(b, b)

    def group(i, _):
        for u in range(4):
            q = u
            b = u % NBUF
            t = i * 4 + u
            gw(b)
            iwd(q)
            ss_(q, b)
            sw(b)
            isx(t + NIDX, q)
            isd(t + NIDX, q)
            iwx((u + NBUF) % NIDX)
            gs((u + NBUF) % NIDX, b)
        return 0

    lax.fori_loop(0, nchunk // 4 - 1, group, 0)

    i = nchunk // 4 - 1
    for u in range(4):
        q = u
        b = u % NBUF
        t = i * 4 + u
        gw(b)
        iwd(q)
        ss_(q, b)
        sw(b)
        if u + NBUF < 4:
            iwx((u + NBUF) % NIDX)
            gs((u + NBUF) % NIDX, b)


def _agg0_body(x_hbm, src_hbm, dst_hbm, z2d_hbm, z1d_hbm,
               agg_hbm, cnt_hbm,
               ix0, ix1, ix2, ix3, id0, id1, id2, id3, ones_v,
               rows0, rows1, acc, cacc,
               six0, six1, six2, six3, sid0, sid1, sid2, sid3,
               sg0, sg1, ss0, ss1, sc0, sc1):
    c = lax.axis_index("c")
    s = lax.axis_index("s")
    w = c * 16 + s
    nchunk = EPAD // 32 // CHUNK

    r0 = s * RPT
    pltpu.sync_copy(z2d_hbm.at[pl.ds(r0, RPT)], acc.at[pl.ds(r0, RPT)])
    pltpu.sync_copy(z1d_hbm.at[pl.ds(r0, RPT)], cacc.at[pl.ds(r0, RPT)])
    o = jnp.ones((16,), jnp.float32)
    for k in range(CHUNK // 16):
        ones_v[pl.ds(k * 16, 16)] = o
    plsc.subcore_barrier()

    scs = [sc0, sc1]
    dstv = [id0, id1, id2, id3]

    def cnt_scatter(q, b):
        pltpu.async_copy(ones_v, cacc.at[dstv[q]], scs[b], add=True)

    def cnt_wait(b):
        pltpu.make_async_copy(ones_v, cacc.at[dstv[0]], scs[b]).wait()

    _edge_pipeline(x_hbm, src_hbm, dst_hbm,
                   src_base=w * (EPAD // 32), dst_base=w * (EPAD // 32),
                   nchunk=nchunk,
                   idxv=[ix0, ix1, ix2, ix3], dstv=dstv,
                   rows=[rows0, rows1], acc=acc,
                   six=[six0, six1, six2, six3],
                   sid=[sid0, sid1, sid2, sid3],
                   sg=[sg0, sg1], ss=[ss0, ss1],
                   extra_scatter=cnt_scatter, extra_wait=cnt_wait)

    plsc.subcore_barrier()
    pltpu.sync_copy(acc.at[pl.ds(r0, RPT)],
                    agg_hbm.at[pl.ds(c * NP + r0, RPT)])
    pltpu.sync_copy(cacc.at[pl.ds(r0, RPT)],
                    cnt_hbm.at[pl.ds(c * NP + r0, RPT)])


_sc_agg0 = pl.kernel(
    _agg0_body,
    out_type=(jax.ShapeDtypeStruct((2 * NP, 128), jnp.float32),
              jax.ShapeDtypeStruct((2 * NP,), jnp.float32)),
    mesh=_mesh,
    scratch_types=[
        pltpu.VMEM((CHUNK,), jnp.int32),
        pltpu.VMEM((CHUNK,), jnp.int32),
        pltpu.VMEM((CHUNK,), jnp.int32),
        pltpu.VMEM((CHUNK,), jnp.int32),
        pltpu.VMEM((CHUNK,), jnp.int32),
        pltpu.VMEM((CHUNK,), jnp.int32),
        pltpu.VMEM((CHUNK,), jnp.int32),
        pltpu.VMEM((CHUNK,), jnp.int32),
        pltpu.VMEM((CHUNK,), jnp.float32),
        pltpu.VMEM((CHUNK, 128), jnp.float32),
        pltpu.VMEM((CHUNK, 128), jnp.float32),
        pltpu.VMEM_SHARED((NP, 128), jnp.float32),
        pltpu.VMEM_SHARED((NP,), jnp.float32),
    ] + [pltpu.SemaphoreType.DMA] * 14,
)


def _agg_body(h_hbm, srcb_hbm, dst_hbm, z2d_hbm,
              agg_hbm,
              ix0, ix1, ix2, ix3, id0, id1, id2, id3,
              rows0, rows1, acc,
              six0, six1, six2, six3, sid0, sid1, sid2, sid3,
              sg0, sg1, ss0, ss1):
    c = lax.axis_index("c")
    s = lax.axis_index("s")
    ept = EPAD // 16
    nchunk = ept // CHUNK

    r0 = s * RPT
    pltpu.sync_copy(z2d_hbm.at[pl.ds(r0, RPT)], acc.at[pl.ds(r0, RPT)])
    plsc.subcore_barrier()

    _edge_pipeline(h_hbm, srcb_hbm, dst_hbm,
                   src_base=c * EPAD + s * ept, dst_base=s * ept,
                   nchunk=nchunk,
                   idxv=[ix0, ix1, ix2, ix3], dstv=[id0, id1, id2, id3],
                   rows=[rows0, rows1], acc=acc,
                   six=[six0, six1, six2, six3],
                   sid=[sid0, sid1, sid2, sid3],
                   sg=[sg0, sg1], ss=[ss0, ss1])

    plsc.subcore_barrier()
    pltpu.sync_copy(acc.at[pl.ds(r0, RPT)],
                    agg_hbm.at[pl.ds(c * NP + r0, RPT)])


_sc_agg = pl.kernel(
    _agg_body,
    out_type=jax.ShapeDtypeStruct((2 * NP, 128), jnp.float32),
    mesh=_mesh,
    scratch_types=[
        pltpu.VMEM((CHUNK,), jnp.int32),
        pltpu.VMEM((CHUNK,), jnp.int32),
        pltpu.VMEM((CHUNK,), jnp.int32),
        pltpu.VMEM((CHUNK,), jnp.int32),
        pltpu.VMEM((CHUNK,), jnp.int32),
        pltpu.VMEM((CHUNK,), jnp.int32),
        pltpu.VMEM((CHUNK,), jnp.int32),
        pltpu.VMEM((CHUNK,), jnp.int32),
        pltpu.VMEM((CHUNK, 128), jnp.float32),
        pltpu.VMEM((CHUNK, 128), jnp.float32),
        pltpu.VMEM_SHARED((NP, 128), jnp.float32),
    ] + [pltpu.SemaphoreType.DMA] * 12,
)


ROW_BLK = 2048


def _tc_layer_body(nparts_in, parts_out,
                   agg_ref, cnt_ref, h_ref, wl_ref, wr_ref, b_ref, o_ref):
    cnt = cnt_ref[0] + cnt_ref[1]
    inv = 1.0 / jnp.maximum(cnt, 1.0)
    acc = jnp.zeros((ROW_BLK, 128), jnp.float32)
    for c in range(2):
        acc = acc + jnp.dot(agg_ref[c] * inv[:, None], wl_ref[c],
                            preferred_element_type=jnp.float32)
    for q in range(nparts_in):
        acc = acc + jnp.dot(h_ref[q], wr_ref[q],
                            preferred_element_type=jnp.float32)
    acc = acc + b_ref[0][None, :]
    out = jnp.maximum(acc, 0.0)
    if parts_out:
        o_ref[...] = out[None]
    else:
        o_ref[...] = out


def _tc_layer(agg, cnt, h_parts, wlt, wrt, b, parts_out):
    p_in = h_parts.shape[0]
    grid = (5, 2)
    if parts_out:
        out_shape = jax.ShapeDtypeStruct((2, NP, 128), jnp.float32)
        out_spec = pl.BlockSpec((1, ROW_BLK, 128), lambda i, p: (p, i, 0))
    else:
        out_shape = jax.ShapeDtypeStruct((N, D), jnp.float32)
        out_spec = pl.BlockSpec((ROW_BLK, 128), lambda i, p: (i, p))
    return pl.pallas_call(
        functools.partial(_tc_layer_body, p_in, parts_out),
        grid=grid,
        in_specs=[
            pl.BlockSpec((2, ROW_BLK, 128), lambda i, p: (0, i, 0)),
            pl.BlockSpec((2, ROW_BLK), lambda i, p: (0, i)),
            pl.BlockSpec((p_in, ROW_BLK, 128), lambda i, p: (0, i, 0)),
            pl.BlockSpec((2, 128, 128), lambda i, p: (0, 0, p)),
            pl.BlockSpec((p_in, 128, 128), lambda i, p: (0, 0, p)),
            pl.BlockSpec((1, 128), lambda i, p: (0, p)),
        ],
        out_specs=out_spec,
        out_shape=out_shape,
    )(agg, cnt, h_parts, wlt, wrt, b)


def kernel(x, edge_index, Wl0, Wr0, b0, Wl1, Wr1, b1, Wl2, Wr2, b2):
    src = edge_index[0]
    dst = edge_index[1]
    npad = EPAD - E
    src_pad = jnp.concatenate([src, jnp.zeros((npad,), jnp.int32)])
    dst_pad = jnp.concatenate([dst, jnp.full((npad,), NP - 1, jnp.int32)])
    srcb = jnp.concatenate([src_pad, src_pad + NP])
    z2d = jnp.zeros((NP, 128), jnp.float32)
    z1d = jnp.zeros((NP,), jnp.float32)

    agg0, cnt = _sc_agg0(x, src_pad, dst_pad, z2d, z1d)
    agg0 = agg0.reshape(2, NP, 128)
    cnt = cnt.reshape(2, NP)
    h1 = _tc_layer(agg0, cnt, x.reshape(1, N, 128),
                   jnp.stack([Wl0.T, Wl0.T]), Wr0.T.reshape(1, 128, D),
                   b0.reshape(1, D), parts_out=True)

    agg1 = _sc_agg(h1.reshape(2 * NP, 128), srcb, dst_pad, z2d)
    h2 = _tc_layer(agg1.reshape(2, NP, 128), cnt, h1,
                   Wl1.T.reshape(2, 128, D), Wr1.T.reshape(2, 128, D),
                   b1.reshape(1, D), parts_out=True)

    agg2 = _sc_agg(h2.reshape(2 * NP, 128), srcb, dst_pad, z2d)
    h3 = _tc_layer(agg2.reshape(2, NP, 128), cnt, h2,
                   Wl2.T.reshape(2, 128, D), Wr2.T.reshape(2, 128, D),
                   b2.reshape(1, D), parts_out=False)

    return h3.reshape(1, N, D)
